# baseline (device time: 169885 ns/iter reference)
import jax
import jax.numpy as jnp
from jax import lax
from jax.experimental import pallas as pl
from jax.experimental.pallas import tpu as pltpu

N_DEV = 4
M_PER = 1024
N_OUT = 2048


def kernel(x, w_mat):
    x = x.astype(jnp.bfloat16)
    w_mat = w_mat.astype(jnp.bfloat16)
    m_tot, _ = x.shape

    def body(x_ref, w_ref, out_ref, comm, y_ref, amax_buf, ss, rs, a_ss, a_rs):
        pos = lax.axis_index("i")
        left = lax.rem(pos + N_DEV - 1, N_DEV)
        right = lax.rem(pos + 1, N_DEV)

        barrier = pltpu.get_barrier_semaphore()
        for nbr in (left, right):
            pl.semaphore_signal(
                barrier, inc=1, device_id=(nbr,),
                device_id_type=pl.DeviceIdType.MESH,
            )
        pl.semaphore_wait(barrier, 2)

        w_loc = w_ref[...]

        def partial(c):
            xc = x_ref[pl.ds(c * M_PER, M_PER), :]
            return jnp.dot(xc, w_loc, preferred_element_type=jnp.float32)

        comm[0, :, :] = partial(lax.rem(pos + N_DEV - 1, N_DEV)).astype(
            jnp.bfloat16
        )

        acc = None
        for h in range(N_DEV - 1):
            rdma = pltpu.make_async_remote_copy(
                src_ref=comm.at[h],
                dst_ref=comm.at[h + 1],
                send_sem=ss.at[h],
                recv_sem=rs.at[h],
                device_id=(right,),
                device_id_type=pl.DeviceIdType.MESH,
            )
            rdma.start()
            p = partial(lax.rem(pos + (N_DEV - 2 - h), N_DEV))
            rdma.wait()
            acc = comm[h + 1, :, :].astype(jnp.float32) + p
            if h < N_DEV - 2:
                comm[h + 1, :, :] = acc.astype(jnp.bfloat16)

        y_ref[...] = acc

        lamax = jnp.max(jnp.abs(acc))
        amax_buf[pl.ds(pos, 1), :] = jnp.full((1, 128), lamax, jnp.float32)

        sends = []
        for k in range(1, N_DEV):
            tgt = lax.rem(pos + k, N_DEV)
            r = pltpu.make_async_remote_copy(
                src_ref=amax_buf.at[pl.ds(pos, 1)],
                dst_ref=amax_buf.at[pl.ds(pos, 1)],
                send_sem=a_ss.at[k - 1],
                recv_sem=a_rs.at[k - 1],
                device_id=(tgt,),
                device_id_type=pl.DeviceIdType.MESH,
            )
            r.start()
            sends.append(r)
        for k in range(1, N_DEV):
            src_pos = lax.rem(pos + N_DEV - k, N_DEV)
            r = pltpu.make_async_remote_copy(
                src_ref=amax_buf.at[pl.ds(pos, 1)],
                dst_ref=amax_buf.at[pl.ds(src_pos, 1)],
                send_sem=a_ss.at[k - 1],
                recv_sem=a_rs.at[k - 1],
                device_id=(left,),
                device_id_type=pl.DeviceIdType.MESH,
            )
            r.wait_recv()
        for r in sends:
            r.wait_send()

        gmax = jnp.max(amax_buf[...])

        inv = 448.0 / gmax
        q = jnp.clip(y_ref[...] * inv, -448.0, 448.0).astype(jnp.float8_e4m3fn)
        out_ref[...] = q.astype(jnp.float32) * (gmax / 448.0)

    return pl.pallas_call(
        body,
        out_shape=jax.ShapeDtypeStruct((M_PER, N_OUT), jnp.float32),
        in_specs=[
            pl.BlockSpec(memory_space=pltpu.VMEM),
            pl.BlockSpec(memory_space=pltpu.VMEM),
        ],
        out_specs=pl.BlockSpec(memory_space=pltpu.VMEM),
        scratch_shapes=[
            pltpu.VMEM((N_DEV, M_PER, N_OUT), jnp.bfloat16),
            pltpu.VMEM((M_PER, N_OUT), jnp.float32),
            pltpu.VMEM((N_DEV, 128), jnp.float32),
            pltpu.SemaphoreType.DMA((N_DEV - 1,)),
            pltpu.SemaphoreType.DMA((N_DEV - 1,)),
            pltpu.SemaphoreType.DMA((N_DEV - 1,)),
            pltpu.SemaphoreType.DMA((N_DEV - 1,)),
        ],
        compiler_params=pltpu.CompilerParams(collective_id=0),
    )(x, w_mat)


# device time: 102100 ns/iter; 1.6639x vs baseline; 1.6639x over previous
import jax
import jax.numpy as jnp
from jax import lax
from jax.experimental import pallas as pl
from jax.experimental.pallas import tpu as pltpu

N_DEV = 4
M_PER = 1024
N_OUT = 2048
HALF = N_OUT // 2


def kernel(x, w_mat):
    x = x.astype(jnp.bfloat16)
    w_mat = w_mat.astype(jnp.bfloat16)

    def body(x_ref, w_ref, out_ref, comm_cw, comm_ccw, y_ref, amax_buf,
             ss_cw, rs_cw, ss_ccw, rs_ccw, a_ss, a_rs):
        pos = lax.axis_index("i")
        left = lax.rem(pos + N_DEV - 1, N_DEV)
        right = lax.rem(pos + 1, N_DEV)

        barrier = pltpu.get_barrier_semaphore()
        for nbr in (left, right):
            pl.semaphore_signal(
                barrier, inc=1, device_id=(nbr,),
                device_id_type=pl.DeviceIdType.MESH,
            )
        pl.semaphore_wait(barrier, 2)

        w_loc = w_ref[...]

        def xc(c):
            return x_ref[pl.ds(c * M_PER, M_PER), :]

        def dot(a, b):
            return jnp.dot(a, b, preferred_element_type=jnp.float32)

        comm_cw[0, :, :] = dot(
            xc(lax.rem(pos + N_DEV - 1, N_DEV)), w_loc[:, :HALF]
        ).astype(jnp.bfloat16)
        comm_ccw[0, :, :] = dot(
            xc(lax.rem(pos + 1, N_DEV)), w_loc[:, HALF:]
        ).astype(jnp.bfloat16)

        acc_cw = acc_ccw = None
        for h in range(N_DEV - 1):
            rd_cw = pltpu.make_async_remote_copy(
                src_ref=comm_cw.at[h],
                dst_ref=comm_cw.at[h + 1],
                send_sem=ss_cw.at[h],
                recv_sem=rs_cw.at[h],
                device_id=(right,),
                device_id_type=pl.DeviceIdType.MESH,
            )
            rd_ccw = pltpu.make_async_remote_copy(
                src_ref=comm_ccw.at[h],
                dst_ref=comm_ccw.at[h + 1],
                send_sem=ss_ccw.at[h],
                recv_sem=rs_ccw.at[h],
                device_id=(left,),
                device_id_type=pl.DeviceIdType.MESH,
            )
            rd_cw.start()
            rd_ccw.start()

            c_cw = lax.rem(pos + (N_DEV - 2 - h), N_DEV)
            c_ccw = lax.rem(pos + 2 + h, N_DEV)
            if h in (0, N_DEV - 2):
                pf = dot(xc(c_cw), w_loc)
                p_cw, p_ccw = pf[:, :HALF], pf[:, HALF:]
            else:
                p_cw = dot(xc(c_cw), w_loc[:, :HALF])
                p_ccw = dot(xc(c_ccw), w_loc[:, HALF:])

            rd_cw.wait()
            rd_ccw.wait()
            acc_cw = comm_cw[h + 1, :, :].astype(jnp.float32) + p_cw
            acc_ccw = comm_ccw[h + 1, :, :].astype(jnp.float32) + p_ccw
            if h < N_DEV - 2:
                comm_cw[h + 1, :, :] = acc_cw.astype(jnp.bfloat16)
                comm_ccw[h + 1, :, :] = acc_ccw.astype(jnp.bfloat16)

        y_ref[:, :HALF] = acc_cw
        y_ref[:, HALF:] = acc_ccw

        lamax = jnp.maximum(
            jnp.max(jnp.abs(acc_cw)), jnp.max(jnp.abs(acc_ccw))
        )
        amax_buf[pl.ds(pos, 1), :] = jnp.full((1, 128), lamax, jnp.float32)

        sends = []
        for k in range(1, N_DEV):
            tgt = lax.rem(pos + k, N_DEV)
            r = pltpu.make_async_remote_copy(
                src_ref=amax_buf.at[pl.ds(pos, 1)],
                dst_ref=amax_buf.at[pl.ds(pos, 1)],
                send_sem=a_ss.at[k - 1],
                recv_sem=a_rs.at[k - 1],
                device_id=(tgt,),
                device_id_type=pl.DeviceIdType.MESH,
            )
            r.start()
            sends.append(r)
        for k in range(1, N_DEV):
            src_pos = lax.rem(pos + N_DEV - k, N_DEV)
            r = pltpu.make_async_remote_copy(
                src_ref=amax_buf.at[pl.ds(pos, 1)],
                dst_ref=amax_buf.at[pl.ds(src_pos, 1)],
                send_sem=a_ss.at[k - 1],
                recv_sem=a_rs.at[k - 1],
                device_id=(left,),
                device_id_type=pl.DeviceIdType.MESH,
            )
            r.wait_recv()
        for r in sends:
            r.wait_send()

        gmax = jnp.max(amax_buf[...])

        inv = 448.0 / gmax
        q = jnp.clip(y_ref[...] * inv, -448.0, 448.0).astype(jnp.float8_e4m3fn)
        out_ref[...] = q.astype(jnp.float32) * (gmax / 448.0)

    return pl.pallas_call(
        body,
        out_shape=jax.ShapeDtypeStruct((M_PER, N_OUT), jnp.float32),
        in_specs=[
            pl.BlockSpec(memory_space=pltpu.VMEM),
            pl.BlockSpec(memory_space=pltpu.VMEM),
        ],
        out_specs=pl.BlockSpec(memory_space=pltpu.VMEM),
        scratch_shapes=[
            pltpu.VMEM((N_DEV, M_PER, HALF), jnp.bfloat16),
            pltpu.VMEM((N_DEV, M_PER, HALF), jnp.bfloat16),
            pltpu.VMEM((M_PER, N_OUT), jnp.float32),
            pltpu.VMEM((N_DEV, 128), jnp.float32),
            pltpu.SemaphoreType.DMA((N_DEV - 1,)),
            pltpu.SemaphoreType.DMA((N_DEV - 1,)),
            pltpu.SemaphoreType.DMA((N_DEV - 1,)),
            pltpu.SemaphoreType.DMA((N_DEV - 1,)),
            pltpu.SemaphoreType.DMA((N_DEV - 1,)),
            pltpu.SemaphoreType.DMA((N_DEV - 1,)),
        ],
        compiler_params=pltpu.CompilerParams(collective_id=0),
    )(x, w_mat)


# device time: 93571 ns/iter; 1.8156x vs baseline; 1.0912x over previous
import jax
import jax.numpy as jnp
from jax import lax
from jax.experimental import pallas as pl
from jax.experimental.pallas import tpu as pltpu

N_DEV = 4
N_HOP = N_DEV - 1
M_PER = 1024
N_OUT = 2048
HALF = N_OUT // 2
SUB = HALF // 2


def kernel(x, w_mat):
    x = x.astype(jnp.bfloat16)
    w_mat = w_mat.astype(jnp.bfloat16)

    def body(x_ref, w_ref, out_ref, comm_cw, comm_ccw, y_ref, amax_buf,
             ss_cw, rs_cw, ss_ccw, rs_ccw, a_ss, a_rs):
        pos = lax.axis_index("i")
        left = lax.rem(pos + N_DEV - 1, N_DEV)
        right = lax.rem(pos + 1, N_DEV)

        barrier = pltpu.get_barrier_semaphore()
        for nbr in (left, right):
            pl.semaphore_signal(
                barrier, inc=1, device_id=(nbr,),
                device_id_type=pl.DeviceIdType.MESH,
            )
        pl.semaphore_wait(barrier, 2)

        w_loc = w_ref[...]
        comm = {"cw": comm_cw, "ccw": comm_ccw}
        ss = {"cw": ss_cw, "ccw": ss_ccw}
        rs = {"cw": rs_cw, "ccw": rs_ccw}
        peer = {"cw": right, "ccw": left}
        col0 = {"cw": 0, "ccw": HALF}

        def xc(c):
            return x_ref[pl.ds(c * M_PER, M_PER), :]

        def psub(c, d, s):
            lo = col0[d] + s * SUB
            return jnp.dot(
                xc(c), w_loc[:, lo:lo + SUB],
                preferred_element_type=jnp.float32,
            )

        def make_rdma(d, h, s):
            return pltpu.make_async_remote_copy(
                src_ref=comm[d].at[h, s],
                dst_ref=comm[d].at[h + 1, s],
                send_sem=ss[d].at[h, s],
                recv_sem=rs[d].at[h, s],
                device_id=(peer[d],),
                device_id_type=pl.DeviceIdType.MESH,
            )

        def c_arr(d, h):
            off = (N_DEV - 2 - h) if d == "cw" else (2 + h)
            return lax.rem(pos + off, N_DEV)

        sends = []

        c_seed = {"cw": lax.rem(pos + N_DEV - 1, N_DEV),
                  "ccw": lax.rem(pos + 1, N_DEV)}
        for s in range(2):
            for d in ("cw", "ccw"):
                comm[d][0, s] = psub(c_seed[d], d, s).astype(jnp.bfloat16)
                r = make_rdma(d, 0, s)
                r.start()
                sends.append(r)

        acc_final = {}
        for h in range(N_HOP):
            p = {(d, s): psub(c_arr(d, h), d, s)
                 for s in range(2) for d in ("cw", "ccw")}
            for s in range(2):
                for d in ("cw", "ccw"):
                    make_rdma(d, h, s).wait_recv()
                    acc = comm[d][h + 1, s].astype(jnp.float32) + p[(d, s)]
                    if h < N_HOP - 1:
                        comm[d][h + 1, s] = acc.astype(jnp.bfloat16)
                        r = make_rdma(d, h + 1, s)
                        r.start()
                        sends.append(r)
                    else:
                        acc_final[(d, s)] = acc

        lamax = jnp.float32(0)
        for (d, s), acc in acc_final.items():
            lo = col0[d] + s * SUB
            y_ref[:, lo:lo + SUB] = acc
            lamax = jnp.maximum(lamax, jnp.max(jnp.abs(acc)))

        amax_buf[pl.ds(pos, 1), :] = jnp.full((1, 128), lamax, jnp.float32)
        for k in range(1, N_DEV):
            tgt = lax.rem(pos + k, N_DEV)
            r = pltpu.make_async_remote_copy(
                src_ref=amax_buf.at[pl.ds(pos, 1)],
                dst_ref=amax_buf.at[pl.ds(pos, 1)],
                send_sem=a_ss.at[k - 1],
                recv_sem=a_rs.at[k - 1],
                device_id=(tgt,),
                device_id_type=pl.DeviceIdType.MESH,
            )
            r.start()
            sends.append(r)
        for k in range(1, N_DEV):
            src_pos = lax.rem(pos + N_DEV - k, N_DEV)
            r = pltpu.make_async_remote_copy(
                src_ref=amax_buf.at[pl.ds(pos, 1)],
                dst_ref=amax_buf.at[pl.ds(src_pos, 1)],
                send_sem=a_ss.at[k - 1],
                recv_sem=a_rs.at[k - 1],
                device_id=(left,),
                device_id_type=pl.DeviceIdType.MESH,
            )
            r.wait_recv()
        for r in sends:
            r.wait_send()

        gmax = jnp.max(amax_buf[...])

        inv = 448.0 / gmax
        q = jnp.clip(y_ref[...] * inv, -448.0, 448.0).astype(jnp.float8_e4m3fn)
        out_ref[...] = q.astype(jnp.float32) * (gmax / 448.0)

    return pl.pallas_call(
        body,
        out_shape=jax.ShapeDtypeStruct((M_PER, N_OUT), jnp.float32),
        in_specs=[
            pl.BlockSpec(memory_space=pltpu.VMEM),
            pl.BlockSpec(memory_space=pltpu.VMEM),
        ],
        out_specs=pl.BlockSpec(memory_space=pltpu.VMEM),
        scratch_shapes=[
            pltpu.VMEM((N_DEV, 2, M_PER, SUB), jnp.bfloat16),
            pltpu.VMEM((N_DEV, 2, M_PER, SUB), jnp.bfloat16),
            pltpu.VMEM((M_PER, N_OUT), jnp.float32),
            pltpu.VMEM((N_DEV, 128), jnp.float32),
            pltpu.SemaphoreType.DMA((N_HOP, 2)),
            pltpu.SemaphoreType.DMA((N_HOP, 2)),
            pltpu.SemaphoreType.DMA((N_HOP, 2)),
            pltpu.SemaphoreType.DMA((N_HOP, 2)),
            pltpu.SemaphoreType.DMA((N_DEV - 1,)),
            pltpu.SemaphoreType.DMA((N_DEV - 1,)),
        ],
        compiler_params=pltpu.CompilerParams(collective_id=0),
    )(x, w_mat)
